# baseline (speedup 1.0000x reference)
import functools

import jax
import jax.numpy as jnp
from jax import lax
from jax.experimental import pallas as pl
from jax.experimental.pallas import tpu as pltpu
from jax.experimental.pallas import tpu_sc as plsc

N_PAD = 10240
NODES_PER_TILE = 640
LANES = 16
BIG = 1e30


def _shift(v, iota, d):
  idx = jnp.maximum(iota - d, 0)
  return jnp.take_along_axis(v, idx, axis=0)


def _seg_scan(vals, keys, iota, op):
  for d in (1, 2, 4, 8):
    kd = _shift(keys, iota, d)
    vd = _shift(vals, iota, d)
    ok = (kd == keys) & (iota >= d)
    vals = jnp.where(ok, op(vals, vd), vals)
  return vals


def _make_kernel_a(n_edges):
  ept = n_edges // 16
  n_vregs = ept // LANES
  mesh = plsc.VectorSubcoreMesh(core_axis_name="c", subcore_axis_name="s")

  @functools.partial(
      pl.kernel,
      out_type=jax.ShapeDtypeStruct((10 * N_PAD,), jnp.float32),
      mesh=mesh,
      compiler_params=pltpu.CompilerParams(needs_layout_passes=False),
      scratch_types=[
          pltpu.VMEM((N_PAD,), jnp.float32),
          pltpu.VMEM((N_PAD,), jnp.float32),
          pltpu.VMEM((N_PAD,), jnp.float32),
          pltpu.VMEM((ept,), jnp.int32),
          pltpu.VMEM((ept,), jnp.int32),
          pltpu.VMEM((N_PAD,), jnp.float32),
          pltpu.VMEM((N_PAD,), jnp.float32),
          pltpu.VMEM((N_PAD,), jnp.float32),
          pltpu.VMEM((N_PAD,), jnp.float32),
          pltpu.VMEM((16 * NODES_PER_TILE,), jnp.float32),
          pltpu.VMEM((NODES_PER_TILE,), jnp.float32),
          pltpu.VMEM_SHARED((16 * 2 * N_PAD,), jnp.float32),
      ],
  )
  def kernel_a(pos_hbm, row_hbm, col_hbm, out_hbm,
               posx, posy, posz, rowv, colv,
               acc0, acc1, acc2, acc3, stage, res, slab):
    c = lax.axis_index("c")
    s = lax.axis_index("s")
    accs = (acc0, acc1, acc2)

    pltpu.sync_copy(pos_hbm.at[pl.ds(0, N_PAD)], posx)
    pltpu.sync_copy(pos_hbm.at[pl.ds(N_PAD, N_PAD)], posy)
    pltpu.sync_copy(pos_hbm.at[pl.ds(2 * N_PAD, N_PAD)], posz)
    pltpu.sync_copy(row_hbm.at[pl.ds(s * ept, ept)], rowv)
    pltpu.sync_copy(col_hbm.at[pl.ds(s * ept, ept)], colv)

    init_val = jnp.where(c == 0, jnp.float32(0.0), jnp.float32(BIG))

    def init_body(i, _):
      iv = jnp.full((LANES,), init_val, jnp.float32)
      zv = jnp.zeros((LANES,), jnp.float32)
      sl = pl.ds(i * LANES, LANES)
      acc0[sl] = iv
      acc1[sl] = iv
      acc2[sl] = iv
      acc3[sl] = zv
      return 0

    lax.fori_loop(0, N_PAD // LANES, init_body, 0)

    iota = lax.iota(jnp.int32, 16)
    ones = jnp.ones((LANES,), jnp.float32)

    def edge_body(j, _):
      sl = pl.ds(j * LANES, LANES)
      col16 = colv[sl]
      row16 = rowv[sl]
      ax = jnp.abs(plsc.load_gather(posx, [row16]) - plsc.load_gather(posx, [col16]))
      ay = jnp.abs(plsc.load_gather(posy, [row16]) - plsc.load_gather(posy, [col16]))
      az = jnp.abs(plsc.load_gather(posz, [row16]) - plsc.load_gather(posz, [col16]))
      colS, sx, sy, sz = lax.sort((col16, ax, ay, az), dimension=0, num_keys=1)
      knext = jnp.take_along_axis(colS, jnp.minimum(iota + 1, 15), axis=0)
      is_last = (colS != knext) | (iota == 15)

      @pl.when(c == 0)
      def _():
        for acc, vals in zip(accs, (sx, sy, sz)):
          m = _seg_scan(vals, colS, iota, jnp.maximum)
          cur = plsc.load_gather(acc, [colS])
          plsc.store_scatter(acc, [colS], jnp.maximum(cur, m), mask=is_last)

      @pl.when(c == 1)
      def _():
        for acc, vals in zip(accs, (sx, sy, sz)):
          m = _seg_scan(vals, colS, iota, jnp.minimum)
          cur = plsc.load_gather(acc, [colS])
          plsc.store_scatter(acc, [colS], jnp.minimum(cur, m), mask=is_last)
        cnt = _seg_scan(ones, colS, iota, jnp.add)
        cur = plsc.load_gather(acc3, [colS])
        plsc.store_scatter(acc3, [colS], cur + cnt, mask=is_last)

      return 0

    lax.fori_loop(0, n_vregs, edge_body, 0)

    n0 = s * NODES_PER_TILE

    def publish(phase):
      a_lo, a_hi = (acc0, acc1) if phase == 0 else (acc2, acc3)
      pltpu.sync_copy(a_lo, slab.at[pl.ds((s * 2 + 0) * N_PAD, N_PAD)])
      pltpu.sync_copy(a_hi, slab.at[pl.ds((s * 2 + 1) * N_PAD, N_PAD)])
      plsc.subcore_barrier()

    def merge_array(a, op):
      for src in range(16):
        pltpu.sync_copy(
            slab.at[pl.ds((src * 2 + a % 2) * N_PAD + n0, NODES_PER_TILE)],
            stage.at[pl.ds(src * NODES_PER_TILE, NODES_PER_TILE)])

      def red_body(i, _):
        sl = i * LANES
        v = stage[pl.ds(sl, LANES)]
        for src in range(1, 16):
          v = op(v, stage[pl.ds(src * NODES_PER_TILE + sl, LANES)])
        res[pl.ds(sl, LANES)] = v
        return 0

      lax.fori_loop(0, NODES_PER_TILE // LANES, red_body, 0)

    def fix_min(_unused):
      def fix_body(i, _):
        sl = pl.ds(i * LANES, LANES)
        v = res[sl]
        res[sl] = jnp.where(v > jnp.float32(9e29), jnp.float32(0.0), v)
        return 0

      lax.fori_loop(0, NODES_PER_TILE // LANES, fix_body, 0)

    for phase in range(2):
      publish(phase)

      @pl.when(c == 0)
      def _():
        for a in (0, 1) if phase == 0 else (2,):
          merge_array(a, jnp.maximum)
          pltpu.sync_copy(res,
                          out_hbm.at[pl.ds(a * N_PAD + n0, NODES_PER_TILE)])

      @pl.when(c == 1)
      def _():
        for a in (0, 1) if phase == 0 else (2,):
          merge_array(a, jnp.minimum)
          fix_min(None)
          pltpu.sync_copy(
              res, out_hbm.at[pl.ds((3 + a) * N_PAD + n0, NODES_PER_TILE)])
        if phase == 1:
          merge_array(3, jnp.add)
          pltpu.sync_copy(res,
                          out_hbm.at[pl.ds(9 * N_PAD + n0, NODES_PER_TILE)])

      plsc.subcore_barrier()

      def bary_body(i, _):
        sl = pl.ds(n0 + i * LANES, LANES)
        b = (posx[sl] + posy[sl] + posz[sl]) * jnp.float32(1.0 / 3.0)
        res[pl.ds(i * LANES, LANES)] = b
        return 0

      lax.fori_loop(0, NODES_PER_TILE // LANES, bary_body, 0)
      for a in (6, 7, 8):
        pltpu.sync_copy(res, out_hbm.at[pl.ds(a * N_PAD + n0, NODES_PER_TILE)])

  return kernel_a


def _make_kernel_c(n_edges, chk, n_chunks):
  ept = n_edges // 16
  mesh = plsc.VectorSubcoreMesh(core_axis_name="c", subcore_axis_name="s")

  @functools.partial(
      pl.kernel,
      out_type=jax.ShapeDtypeStruct((2, N_PAD, 128), jnp.float32),
      mesh=mesh,
      compiler_params=pltpu.CompilerParams(needs_layout_passes=False,
                                           use_tc_tiling_on_sc=False),
      scratch_types=[
          pltpu.VMEM((n_chunks, chk), jnp.int32),
          pltpu.VMEM((n_chunks, chk), jnp.int32),
          pltpu.VMEM((chk, 128), jnp.float32),
          pltpu.VMEM((chk, 128), jnp.float32),
          pltpu.VMEM((chk, 128), jnp.float32),
          pltpu.VMEM((chk, 128), jnp.float32),
          pltpu.SemaphoreType.DMA,
          pltpu.SemaphoreType.DMA,
          pltpu.SemaphoreType.DMA,
          pltpu.SemaphoreType.DMA,
          pltpu.VMEM_SHARED((N_PAD, 128), jnp.float32),
      ],
  )
  def kernel_c(prow_hbm, pcol_hbm, row_hbm, col_hbm, out_hbm,
               idxr, idxc, bufr0, bufc0, bufr1, bufc1,
               semg0, semg1, sems0, sems1, hacc):
    c = lax.axis_index("c")
    s = lax.axis_index("s")
    tbl_r = prow_hbm.at[c]
    tbl_c = pcol_hbm.at[c]
    bufs = ((bufr0, bufc0, semg0, sems0), (bufr1, bufc1, semg1, sems1))

    pltpu.sync_copy(row_hbm.at[s], idxr)
    pltpu.sync_copy(col_hbm.at[s], idxc)

    def zfill_body(i, _):
      bufr0[i // 8, pl.ds((i % 8) * LANES, LANES)] = jnp.zeros((LANES,),
                                                               jnp.float32)
      return 0

    lax.fori_loop(0, chk * 8, zfill_body, 0)

    def zcopy_body(i, _):
      pltpu.sync_copy(bufr0, hacc.at[pl.ds(s * NODES_PER_TILE + i * chk, chk)])
      return 0

    lax.fori_loop(0, NODES_PER_TILE // chk, zcopy_body, 0)
    plsc.subcore_barrier()

    pltpu.async_copy(tbl_r.at[idxr.at[0]], bufr0, semg0)
    pltpu.async_copy(tbl_c.at[idxc.at[0]], bufc0, semg0)

    def round_body(kk, _):
      for p in (0, 1):
        br, bc, semg, _sems = bufs[p]
        qr, qc, qsemg, qsems = bufs[1 - p]
        k = kk * 2 + p

        @pl.when(k + 1 < n_chunks)
        def _():
          pltpu.async_copy(tbl_r.at[idxr.at[k + 1]], qr, qsemg)
          pltpu.async_copy(tbl_c.at[idxc.at[k + 1]], qc, qsemg)

        pltpu.make_async_copy(tbl_r.at[idxr.at[k]], br, semg).wait()
        pltpu.make_async_copy(tbl_c.at[idxc.at[k]], bc, semg).wait()

        @plsc.parallel_loop(0, chk, unroll=8)
        def _(r):
          for j in range(8):
            sl = pl.ds(j * LANES, LANES)
            br[r, sl] = jnp.maximum(br[r, sl] - bc[r, sl], jnp.float32(0.0))

        pltpu.sync_copy(br, hacc.at[idxc.at[k]], add=True)
      return 0

    lax.fori_loop(0, n_chunks // 2, round_body, 0)
    plsc.subcore_barrier()

    pltpu.sync_copy(hacc.at[pl.ds(s * NODES_PER_TILE, NODES_PER_TILE)],
                    out_hbm.at[c, pl.ds(s * NODES_PER_TILE, NODES_PER_TILE)])

  return kernel_c


def _matmul1(x, gp, w1x, w1gp, b1r, n_nodes, mb):
  nm = n_nodes // mb

  def body(x_ref, g_ref, wx_ref, wg_ref, b_ref, prow_ref, pcol_ref):
    acc = jnp.dot(x_ref[...], wx_ref[...], preferred_element_type=jnp.float32)
    acc += jnp.dot(g_ref[...], wg_ref[...], preferred_element_type=jnp.float32)
    pcol_ref[0] = acc
    prow_ref[0] = acc + b_ref[0]

  return pl.pallas_call(
      body,
      grid=(2, nm),
      in_specs=[
          pl.BlockSpec((mb, 256), lambda c, m: (m, 0)),
          pl.BlockSpec((mb, 16), lambda c, m: (m, 0)),
          pl.BlockSpec((256, 128), lambda c, m: (0, c)),
          pl.BlockSpec((16, 128), lambda c, m: (0, c)),
          pl.BlockSpec((1, 1, 128), lambda c, m: (c, 0, 0)),
      ],
      out_specs=[
          pl.BlockSpec((1, mb, 128), lambda c, m: (c, m, 0)),
          pl.BlockSpec((1, mb, 128), lambda c, m: (c, m, 0)),
      ],
      out_shape=[
          jax.ShapeDtypeStruct((2, n_nodes, 128), jnp.float32),
          jax.ShapeDtypeStruct((2, n_nodes, 128), jnp.float32),
      ],
  )(x, gp, w1x, w1gp, b1r)


def _matmul2(h2, degr, w2r, b2r, n_nodes, mb):
  nm = n_nodes // mb

  def body(h_ref, d_ref, w_ref, b_ref, out_ref):
    acc = jnp.dot(h_ref[0], w_ref[0], preferred_element_type=jnp.float32)
    acc += jnp.dot(h_ref[1], w_ref[1], preferred_element_type=jnp.float32)
    out_ref[...] = acc + d_ref[0, 0][:, None] * b_ref[...]

  return pl.pallas_call(
      body,
      grid=(nm,),
      in_specs=[
          pl.BlockSpec((2, mb, 128), lambda m: (0, m, 0)),
          pl.BlockSpec((1, 1, mb), lambda m: (m, 0, 0)),
          pl.BlockSpec((2, 128, 256), lambda m: (0, 0, 0)),
          pl.BlockSpec((1, 256), lambda m: (0, 0)),
      ],
      out_specs=pl.BlockSpec((mb, 256), lambda m: (m, 0)),
      out_shape=jax.ShapeDtypeStruct((n_nodes, 256), jnp.float32),
  )(h2, degr, w2r, b2r)


def kernel(x, pos, edge_index, W1, b1, W2, b2):
  n, cin = x.shape
  e = edge_index.shape[1]
  row = edge_index[0].astype(jnp.int32)
  col = edge_index[1].astype(jnp.int32)

  pos_t = jnp.zeros((3, N_PAD), jnp.float32).at[:, :n].set(pos.T).reshape(-1)
  ka = _make_kernel_a(e)
  stats = ka(pos_t, row, col).reshape(10, N_PAD)

  g = stats[:9, :n].T
  gp = jnp.concatenate([g, jnp.zeros((n, 7), jnp.float32)], axis=1)
  w1g = jnp.concatenate([W1[:9], jnp.zeros((7, 256), jnp.float32)], axis=0)
  w1x = W1[9:]
  b1r = b1.reshape(2, 1, 128)
  p_row, p_col = _matmul1(x, gp, w1x, w1g, b1r, n, 400)

  chk = 40
  ept = e // 16
  n_chunks = ept // chk
  row_r = row.reshape(16, n_chunks, chk)
  col_r = col.reshape(16, n_chunks, chk)
  kc = _make_kernel_c(e, chk, n_chunks)
  h2 = kc(p_row, p_col, row_r, col_r)

  deg = stats[9, :n]
  degr = deg.reshape(25, 1, 400)
  w2r = W2.reshape(2, 128, 256)
  b2r = b2.reshape(1, 256)
  out = _matmul2(h2, degr, w2r, b2r, n, 400)
  return out


# speedup vs baseline: 1.0043x; 1.0043x over previous
import functools

import jax
import jax.numpy as jnp
from jax import lax
from jax.experimental import pallas as pl
from jax.experimental.pallas import tpu as pltpu
from jax.experimental.pallas import tpu_sc as plsc

N_PAD = 10240
NODES_PER_TILE = 640
LANES = 16
BIG = 1e30


def _shift(v, iota, d):
  idx = jnp.maximum(iota - d, 0)
  return jnp.take_along_axis(v, idx, axis=0)


def _seg_scan(vals, keys, iota, op):
  for d in (1, 2, 4, 8):
    kd = _shift(keys, iota, d)
    vd = _shift(vals, iota, d)
    ok = (kd == keys) & (iota >= d)
    vals = jnp.where(ok, op(vals, vd), vals)
  return vals


def _make_kernel_a(n_edges):
  ept = n_edges // 16
  n_vregs = ept // LANES
  mesh = plsc.VectorSubcoreMesh(core_axis_name="c", subcore_axis_name="s")

  @functools.partial(
      pl.kernel,
      out_type=jax.ShapeDtypeStruct((10 * N_PAD,), jnp.float32),
      mesh=mesh,
      compiler_params=pltpu.CompilerParams(needs_layout_passes=False),
      scratch_types=[
          pltpu.VMEM((N_PAD,), jnp.float32),
          pltpu.VMEM((N_PAD,), jnp.float32),
          pltpu.VMEM((N_PAD,), jnp.float32),
          pltpu.VMEM((ept,), jnp.int32),
          pltpu.VMEM((ept,), jnp.int32),
          pltpu.VMEM((N_PAD,), jnp.float32),
          pltpu.VMEM((N_PAD,), jnp.float32),
          pltpu.VMEM((N_PAD,), jnp.float32),
          pltpu.VMEM((N_PAD,), jnp.float32),
          pltpu.VMEM((16 * NODES_PER_TILE,), jnp.float32),
          pltpu.VMEM((NODES_PER_TILE,), jnp.float32),
          pltpu.VMEM_SHARED((16 * 2 * N_PAD,), jnp.float32),
      ],
  )
  def kernel_a(pos_hbm, row_hbm, col_hbm, out_hbm,
               posx, posy, posz, rowv, colv,
               acc0, acc1, acc2, acc3, stage, res, slab):
    c = lax.axis_index("c")
    s = lax.axis_index("s")
    accs = (acc0, acc1, acc2)

    pltpu.sync_copy(pos_hbm.at[pl.ds(0, N_PAD)], posx)
    pltpu.sync_copy(pos_hbm.at[pl.ds(N_PAD, N_PAD)], posy)
    pltpu.sync_copy(pos_hbm.at[pl.ds(2 * N_PAD, N_PAD)], posz)
    pltpu.sync_copy(row_hbm.at[pl.ds(s * ept, ept)], rowv)
    pltpu.sync_copy(col_hbm.at[pl.ds(s * ept, ept)], colv)

    init_val = jnp.where(c == 0, jnp.float32(0.0), jnp.float32(BIG))

    def init_body(i, _):
      iv = jnp.full((LANES,), init_val, jnp.float32)
      zv = jnp.zeros((LANES,), jnp.float32)
      sl = pl.ds(i * LANES, LANES)
      acc0[sl] = iv
      acc1[sl] = iv
      acc2[sl] = iv
      acc3[sl] = zv
      return 0

    lax.fori_loop(0, N_PAD // LANES, init_body, 0)

    iota = lax.iota(jnp.int32, 16)
    ones = jnp.ones((LANES,), jnp.float32)

    def edge_body(j, _):
      sl = pl.ds(j * LANES, LANES)
      col16 = colv[sl]
      row16 = rowv[sl]
      ax = jnp.abs(plsc.load_gather(posx, [row16]) - plsc.load_gather(posx, [col16]))
      ay = jnp.abs(plsc.load_gather(posy, [row16]) - plsc.load_gather(posy, [col16]))
      az = jnp.abs(plsc.load_gather(posz, [row16]) - plsc.load_gather(posz, [col16]))
      colS, sx, sy, sz = lax.sort((col16, ax, ay, az), dimension=0, num_keys=1)
      knext = jnp.take_along_axis(colS, jnp.minimum(iota + 1, 15), axis=0)
      is_last = (colS != knext) | (iota == 15)

      @pl.when(c == 0)
      def _():
        for acc, vals in zip(accs, (sx, sy, sz)):
          m = _seg_scan(vals, colS, iota, jnp.maximum)
          cur = plsc.load_gather(acc, [colS])
          plsc.store_scatter(acc, [colS], jnp.maximum(cur, m), mask=is_last)

      @pl.when(c == 1)
      def _():
        for acc, vals in zip(accs, (sx, sy, sz)):
          m = _seg_scan(vals, colS, iota, jnp.minimum)
          cur = plsc.load_gather(acc, [colS])
          plsc.store_scatter(acc, [colS], jnp.minimum(cur, m), mask=is_last)
        cnt = _seg_scan(ones, colS, iota, jnp.add)
        cur = plsc.load_gather(acc3, [colS])
        plsc.store_scatter(acc3, [colS], cur + cnt, mask=is_last)

      return 0

    lax.fori_loop(0, n_vregs, edge_body, 0)

    n0 = s * NODES_PER_TILE

    def publish(phase):
      a_lo, a_hi = (acc0, acc1) if phase == 0 else (acc2, acc3)
      pltpu.sync_copy(a_lo, slab.at[pl.ds((s * 2 + 0) * N_PAD, N_PAD)])
      pltpu.sync_copy(a_hi, slab.at[pl.ds((s * 2 + 1) * N_PAD, N_PAD)])
      plsc.subcore_barrier()

    def merge_array(a, op):
      for src in range(16):
        pltpu.sync_copy(
            slab.at[pl.ds((src * 2 + a % 2) * N_PAD + n0, NODES_PER_TILE)],
            stage.at[pl.ds(src * NODES_PER_TILE, NODES_PER_TILE)])

      def red_body(i, _):
        sl = i * LANES
        v = stage[pl.ds(sl, LANES)]
        for src in range(1, 16):
          v = op(v, stage[pl.ds(src * NODES_PER_TILE + sl, LANES)])
        res[pl.ds(sl, LANES)] = v
        return 0

      lax.fori_loop(0, NODES_PER_TILE // LANES, red_body, 0)

    def fix_min(_unused):
      def fix_body(i, _):
        sl = pl.ds(i * LANES, LANES)
        v = res[sl]
        res[sl] = jnp.where(v > jnp.float32(9e29), jnp.float32(0.0), v)
        return 0

      lax.fori_loop(0, NODES_PER_TILE // LANES, fix_body, 0)

    for phase in range(2):
      publish(phase)

      @pl.when(c == 0)
      def _():
        for a in (0, 1) if phase == 0 else (2,):
          merge_array(a, jnp.maximum)
          pltpu.sync_copy(res,
                          out_hbm.at[pl.ds(a * N_PAD + n0, NODES_PER_TILE)])

      @pl.when(c == 1)
      def _():
        for a in (0, 1) if phase == 0 else (2,):
          merge_array(a, jnp.minimum)
          fix_min(None)
          pltpu.sync_copy(
              res, out_hbm.at[pl.ds((3 + a) * N_PAD + n0, NODES_PER_TILE)])
        if phase == 1:
          merge_array(3, jnp.add)
          pltpu.sync_copy(res,
                          out_hbm.at[pl.ds(9 * N_PAD + n0, NODES_PER_TILE)])

      plsc.subcore_barrier()

      def bary_body(i, _):
        sl = pl.ds(n0 + i * LANES, LANES)
        b = (posx[sl] + posy[sl] + posz[sl]) * jnp.float32(1.0 / 3.0)
        res[pl.ds(i * LANES, LANES)] = b
        return 0

      lax.fori_loop(0, NODES_PER_TILE // LANES, bary_body, 0)
      for a in (6, 7, 8):
        pltpu.sync_copy(res, out_hbm.at[pl.ds(a * N_PAD + n0, NODES_PER_TILE)])

  return kernel_a


def _make_kernel_c(n_edges, chk, n_chunks):
  ept = n_edges // 16
  mesh = plsc.VectorSubcoreMesh(core_axis_name="c", subcore_axis_name="s")

  @functools.partial(
      pl.kernel,
      out_type=jax.ShapeDtypeStruct((2, N_PAD, 128), jnp.float32),
      mesh=mesh,
      compiler_params=pltpu.CompilerParams(needs_layout_passes=False,
                                           use_tc_tiling_on_sc=False),
      scratch_types=[
          pltpu.VMEM((n_chunks, chk), jnp.int32),
          pltpu.VMEM((n_chunks, chk), jnp.int32),
          pltpu.VMEM((chk, 128), jnp.float32),
          pltpu.VMEM((chk, 128), jnp.float32),
          pltpu.VMEM((chk, 128), jnp.float32),
          pltpu.VMEM((chk, 128), jnp.float32),
          pltpu.SemaphoreType.DMA,
          pltpu.SemaphoreType.DMA,
          pltpu.SemaphoreType.DMA,
          pltpu.SemaphoreType.DMA,
          pltpu.VMEM_SHARED((N_PAD, 128), jnp.float32),
      ],
  )
  def kernel_c(prow_hbm, pcol_hbm, row_hbm, col_hbm, out_hbm,
               idxr, idxc, bufr0, bufc0, bufr1, bufc1,
               semg0, semg1, sems0, sems1, hacc):
    c = lax.axis_index("c")
    s = lax.axis_index("s")
    tbl_r = prow_hbm.at[c]
    tbl_c = pcol_hbm.at[c]
    bufs = ((bufr0, bufc0, semg0, sems0), (bufr1, bufc1, semg1, sems1))

    pltpu.sync_copy(row_hbm.at[s], idxr)
    pltpu.sync_copy(col_hbm.at[s], idxc)

    def zfill_body(i, _):
      bufr0[i // 8, pl.ds((i % 8) * LANES, LANES)] = jnp.zeros((LANES,),
                                                               jnp.float32)
      return 0

    lax.fori_loop(0, chk * 8, zfill_body, 0)

    def zcopy_body(i, _):
      pltpu.sync_copy(bufr0, hacc.at[pl.ds(s * NODES_PER_TILE + i * chk, chk)])
      return 0

    lax.fori_loop(0, NODES_PER_TILE // chk, zcopy_body, 0)
    plsc.subcore_barrier()

    pltpu.async_copy(tbl_r.at[idxr.at[0]], bufr0, semg0)
    pltpu.async_copy(tbl_c.at[idxc.at[0]], bufc0, semg0)

    def round_body(kk, _):
      for p in (0, 1):
        br, bc, semg, _sems = bufs[p]
        qr, qc, qsemg, qsems = bufs[1 - p]
        k = kk * 2 + p

        @pl.when(k + 1 < n_chunks)
        def _():
          pltpu.async_copy(tbl_r.at[idxr.at[k + 1]], qr, qsemg)
          pltpu.async_copy(tbl_c.at[idxc.at[k + 1]], qc, qsemg)

        pltpu.make_async_copy(tbl_r.at[idxr.at[k]], br, semg).wait()
        pltpu.make_async_copy(tbl_c.at[idxc.at[k]], bc, semg).wait()

        @plsc.parallel_loop(0, chk, unroll=4)
        def _(r):
          for j in range(8):
            sl = pl.ds(j * LANES, LANES)
            br[r, sl] = jnp.maximum(br[r, sl] - bc[r, sl], jnp.float32(0.0))

        pltpu.sync_copy(br, hacc.at[idxc.at[k]], add=True)
      return 0

    lax.fori_loop(0, n_chunks // 2, round_body, 0)
    plsc.subcore_barrier()

    pltpu.sync_copy(hacc.at[pl.ds(s * NODES_PER_TILE, NODES_PER_TILE)],
                    out_hbm.at[c, pl.ds(s * NODES_PER_TILE, NODES_PER_TILE)])

  return kernel_c


def _matmul1(x, gp, w1x, w1gp, b1r, n_nodes, mb):
  nm = n_nodes // mb

  def body(x_ref, g_ref, wx_ref, wg_ref, b_ref, prow_ref, pcol_ref):
    acc = jnp.dot(x_ref[...], wx_ref[...], preferred_element_type=jnp.float32)
    acc += jnp.dot(g_ref[...], wg_ref[...], preferred_element_type=jnp.float32)
    pcol_ref[0] = acc
    prow_ref[0] = acc + b_ref[0]

  return pl.pallas_call(
      body,
      grid=(2, nm),
      in_specs=[
          pl.BlockSpec((mb, 256), lambda c, m: (m, 0)),
          pl.BlockSpec((mb, 16), lambda c, m: (m, 0)),
          pl.BlockSpec((256, 128), lambda c, m: (0, c)),
          pl.BlockSpec((16, 128), lambda c, m: (0, c)),
          pl.BlockSpec((1, 1, 128), lambda c, m: (c, 0, 0)),
      ],
      out_specs=[
          pl.BlockSpec((1, mb, 128), lambda c, m: (c, m, 0)),
          pl.BlockSpec((1, mb, 128), lambda c, m: (c, m, 0)),
      ],
      out_shape=[
          jax.ShapeDtypeStruct((2, n_nodes, 128), jnp.float32),
          jax.ShapeDtypeStruct((2, n_nodes, 128), jnp.float32),
      ],
  )(x, gp, w1x, w1gp, b1r)


def _matmul2(h2, degr, w2r, b2r, n_nodes, mb):
  nm = n_nodes // mb

  def body(h_ref, d_ref, w_ref, b_ref, out_ref):
    acc = jnp.dot(h_ref[0], w_ref[0], preferred_element_type=jnp.float32)
    acc += jnp.dot(h_ref[1], w_ref[1], preferred_element_type=jnp.float32)
    out_ref[...] = acc + d_ref[0, 0][:, None] * b_ref[...]

  return pl.pallas_call(
      body,
      grid=(nm,),
      in_specs=[
          pl.BlockSpec((2, mb, 128), lambda m: (0, m, 0)),
          pl.BlockSpec((1, 1, mb), lambda m: (m, 0, 0)),
          pl.BlockSpec((2, 128, 256), lambda m: (0, 0, 0)),
          pl.BlockSpec((1, 256), lambda m: (0, 0)),
      ],
      out_specs=pl.BlockSpec((mb, 256), lambda m: (m, 0)),
      out_shape=jax.ShapeDtypeStruct((n_nodes, 256), jnp.float32),
  )(h2, degr, w2r, b2r)


def kernel(x, pos, edge_index, W1, b1, W2, b2):
  n, cin = x.shape
  e = edge_index.shape[1]
  row = edge_index[0].astype(jnp.int32)
  col = edge_index[1].astype(jnp.int32)

  pos_t = jnp.zeros((3, N_PAD), jnp.float32).at[:, :n].set(pos.T).reshape(-1)
  ka = _make_kernel_a(e)
  stats = ka(pos_t, row, col).reshape(10, N_PAD)

  g = stats[:9, :n].T
  gp = jnp.concatenate([g, jnp.zeros((n, 7), jnp.float32)], axis=1)
  w1g = jnp.concatenate([W1[:9], jnp.zeros((7, 256), jnp.float32)], axis=0)
  w1x = W1[9:]
  b1r = b1.reshape(2, 1, 128)
  p_row, p_col = _matmul1(x, gp, w1x, w1g, b1r, n, 400)

  chk = 40
  ept = e // 16
  n_chunks = ept // chk
  row_r = row.reshape(16, n_chunks, chk)
  col_r = col.reshape(16, n_chunks, chk)
  kc = _make_kernel_c(e, chk, n_chunks)
  h2 = kc(p_row, p_col, row_r, col_r)

  deg = stats[9, :n]
  degr = deg.reshape(25, 1, 400)
  w2r = W2.reshape(2, 128, 256)
  b2r = b2.reshape(1, 256)
  out = _matmul2(h2, degr, w2r, b2r, n, 400)
  return out
